# baseline (speedup 1.0000x reference)
import functools

import jax
import jax.numpy as jnp
from jax import lax
from jax.experimental import pallas as pl
from jax.experimental.pallas import tpu as pltpu
from jax.experimental.pallas import tpu_sc as plsc

N = 10000
NP = 10240
E = 320000
D = 128
NT = 32
EPT = E // NT
CH = 128
NCHUNK = NP // CH
PAD_COL = NP - 1
SCH = 16

def _conv_body(gt_hbm, row_hbm, col_hbm, w_hbm, out_hbm,
               acc, rowb, colb, wb, bufa, bufb, semA, semB):
    c = lax.axis_index("c")
    s = lax.axis_index("s")
    t = c * 16 + s
    rows_per_tile = NP // 16
    zeros16 = jnp.zeros((16,), jnp.float32)

    def _zero_bufa():
        def _z(i, _):
            for sub in range(8):
                bufa[i, pl.ds(sub * 16, 16)] = zeros16
            return 0
        lax.fori_loop(0, CH, _z, 0)

    def _scale(buf, q):
        def _g(g, _):
            wvec = wb[pl.ds(q * CH + g * 16, 16)]
            for i in range(16):
                wv = wvec[i]
                e = g * 16 + i
                for sub in range(8):
                    sl = pl.ds(sub * 16, 16)
                    buf[e, sl] = buf[e, sl] * wv
            return 0

        lax.fori_loop(0, CH // 16, _g, 0)

    for k in range(3):
        _zero_bufa()
        for z in range(rows_per_tile // CH):
            pltpu.sync_copy(bufa, acc.at[pl.ds(s * rows_per_tile + z * CH, CH)])
        plsc.subcore_barrier()

        def _sc_body(sc, _):
            pltpu.sync_copy(row_hbm.at[k, t, pl.ds(sc * SCH, SCH)], rowb)
            pltpu.sync_copy(col_hbm.at[k, t, pl.ds(sc * SCH, SCH)], colb)
            pltpu.sync_copy(w_hbm.at[k, t, pl.ds(sc * SCH * CH, SCH * CH)], wb)

            def _pair(j, _):
                qa = 2 * j
                qb = 2 * j + 1
                cpa = pltpu.async_copy(gt_hbm.at[rowb.at[qa]], bufa, semA)
                cpb = pltpu.async_copy(gt_hbm.at[rowb.at[qb]], bufb, semB)
                cpa.wait()
                _scale(bufa, qa)
                pltpu.sync_copy(bufa, acc.at[colb.at[qa]], add=True)
                cpb.wait()
                _scale(bufb, qb)
                pltpu.sync_copy(bufb, acc.at[colb.at[qb]], add=True)
                return 0

            lax.fori_loop(0, SCH // 2, _pair, 0)
            return 0

        lax.fori_loop(0, NCHUNK // SCH, _sc_body, 0)
        plsc.subcore_barrier()
        for z in range(rows_per_tile // CH):
            off = s * rows_per_tile + z * CH
            pltpu.sync_copy(acc.at[pl.ds(off, CH)],
                            out_hbm.at[c, k, pl.ds(off, CH)])


def _mm1_body(x_ref, w1t_ref, sdeg_ref, g0_ref, g1_ref, g2_ref, dinv_ref):
    h = jnp.dot(x_ref[...], w1t_ref[...])
    gs = (g0_ref, g1_ref, g2_ref)
    dcols = []
    for k in range(3):
        deg = sdeg_ref[0, k, :, 0:1] + sdeg_ref[1, k, :, 0:1] + 1.0
        dinv = jnp.where(deg > 0, lax.rsqrt(deg), 0.0)
        dcols.append(dinv)
        gs[k][...] = h * dinv
    dinv_ref[...] = jnp.concatenate(
        dcols + [jnp.zeros((dcols[0].shape[0], 5), dcols[0].dtype)], axis=1)


def _mm2_body(S_ref, g0_ref, g1_ref, g2_ref, dinv_ref, b_ref, w2t_ref,
              o0_ref, o1_ref, o2_ref):
    dinv = dinv_ref[...]
    b = b_ref[...]
    gs = (g0_ref[...], g1_ref[...], g2_ref[...])
    h2 = None
    for k in range(3):
        a = jnp.maximum(
            dinv[:, k:k + 1] * (S_ref[0, k] + S_ref[1, k] + gs[k]) + b, 0.0)
        p = jnp.dot(a, w2t_ref[pl.ds(k * D, D), :])
        h2 = p if h2 is None else h2 + p
    o0_ref[...] = h2 * dinv[:, 0:1]
    o1_ref[...] = h2 * dinv[:, 1:2]
    o2_ref[...] = h2 * dinv[:, 2:3]


def _mm3_body(Q_ref, g0_ref, g1_ref, g2_ref, dinv_ref, b_ref, wct_ref,
              bc_ref, out_ref):
    dinv = dinv_ref[...]
    b = b_ref[...]
    gs = (g0_ref[...], g1_ref[...], g2_ref[...])
    logits = None
    for k in range(3):
        a = jnp.maximum(
            dinv[:, k:k + 1] * (Q_ref[0, k] + Q_ref[1, k] + gs[k]) + b, 0.0)
        p = jnp.dot(a, wct_ref[pl.ds(k * D, D), :])
        logits = p if logits is None else logits + p
    logits = logits + bc_ref[...]
    m = jnp.max(logits, axis=1, keepdims=True)
    sh = logits - m
    lse = jnp.log(jnp.sum(jnp.exp(sh), axis=1, keepdims=True))
    out_ref[...] = sh - lse


_SC_KERNELS = None


def _get_sc_kernels():
    global _SC_KERNELS
    if _SC_KERNELS is None:
        mesh = plsc.VectorSubcoreMesh(core_axis_name="c", subcore_axis_name="s")
        conv = functools.partial(
            pl.kernel,
            mesh=mesh,
            out_type=jax.ShapeDtypeStruct((2, 3, NP, D), jnp.float32),
            scratch_types=[
                pltpu.VMEM_SHARED((NP, D), jnp.float32),
                pltpu.VMEM((SCH, CH), jnp.int32),
                pltpu.VMEM((SCH, CH), jnp.int32),
                pltpu.VMEM((SCH * CH,), jnp.float32),
                pltpu.VMEM((CH, D), jnp.float32),
                pltpu.VMEM((CH, D), jnp.float32),
                pltpu.SemaphoreType.DMA,
                pltpu.SemaphoreType.DMA,
            ],
        )(_conv_body)
        _SC_KERNELS = conv
    return _SC_KERNELS


BLK = 1024
GRID = NP // BLK


def _row_spec(width):
    return pl.BlockSpec((BLK, width), lambda i: (i, 0))


def _full_spec(shape):
    return pl.BlockSpec(shape, lambda i: tuple(0 for _ in shape))


def kernel(x, edge_index, edge_in, edge_out, in_w, out_w, W1, b1, W2, b2, Wc, bc):
    f32 = jnp.float32
    i32 = jnp.int32

    ones_w = jnp.ones((E,), f32)
    rows = jnp.stack([edge_index[0], edge_in[0], edge_out[0]]).astype(i32)
    cols = jnp.stack([edge_index[1], edge_in[1], edge_out[1]]).astype(i32)
    ws = jnp.stack([ones_w, in_w.astype(f32), out_w.astype(f32)])
    rows = rows + (jnp.arange(3, dtype=i32) * NP)[:, None]

    pad = NT * NCHUNK * CH - E
    rows_p = jnp.pad(rows.reshape(3, NT, EPT), ((0, 0), (0, 0), (0, pad // NT)),
                     constant_values=0).reshape(3, NT, NCHUNK, CH)
    cols_p = jnp.pad(cols.reshape(3, NT, EPT), ((0, 0), (0, 0), (0, pad // NT)),
                     constant_values=PAD_COL).reshape(3, NT, NCHUNK, CH)
    ws_p = jnp.pad(ws.reshape(3, NT, EPT), ((0, 0), (0, 0), (0, pad // NT)),
                   constant_values=0.0).reshape(3, NT, NCHUNK, CH)

    x_p = jnp.pad(x.astype(f32), ((0, NP - N), (0, 0)))
    W1T = W1.astype(f32).T
    W2T = W2.astype(f32).T
    WcT = Wc.astype(f32).T
    OUT = Wc.shape[0]
    bc2 = bc.astype(f32)[None, :]

    ws_flat = ws_p.reshape(3, NT, NCHUNK * CH)

    _conv_kernel = _get_sc_kernels()
    ones_tbl = jnp.ones((8, D), f32)
    rows_zero = jnp.zeros_like(rows_p)
    Sdeg = _conv_kernel(ones_tbl, rows_zero, cols_p, ws_flat)

    g0, g1, g2, dinv = pl.pallas_call(
        _mm1_body,
        grid=(GRID,),
        in_specs=[
            _row_spec(D),
            _full_spec((D, D)),
            pl.BlockSpec((2, 3, BLK, D), lambda i: (0, 0, i, 0)),
        ],
        out_specs=[_row_spec(D), _row_spec(D), _row_spec(D), _row_spec(8)],
        out_shape=[
            jax.ShapeDtypeStruct((NP, D), f32),
            jax.ShapeDtypeStruct((NP, D), f32),
            jax.ShapeDtypeStruct((NP, D), f32),
            jax.ShapeDtypeStruct((NP, 8), f32),
        ],
    )(x_p, W1T, Sdeg)

    gt = jnp.concatenate([g0, g1, g2], axis=0)

    S = _conv_kernel(gt, rows_p, cols_p, ws_flat)

    h0, h1, h2 = pl.pallas_call(
        _mm2_body,
        grid=(GRID,),
        in_specs=[
            pl.BlockSpec((2, 3, BLK, D), lambda i: (0, 0, i, 0)),
            _row_spec(D), _row_spec(D), _row_spec(D), _row_spec(8),
            _full_spec((1, D)),
            _full_spec((3 * D, D)),
        ],
        out_specs=[_row_spec(D), _row_spec(D), _row_spec(D)],
        out_shape=[
            jax.ShapeDtypeStruct((NP, D), f32),
            jax.ShapeDtypeStruct((NP, D), f32),
            jax.ShapeDtypeStruct((NP, D), f32),
        ],
    )(S, g0, g1, g2, dinv, b1.astype(f32), W2T)

    gt2 = jnp.concatenate([h0, h1, h2], axis=0)

    Q = _conv_kernel(gt2, rows_p, cols_p, ws_flat)

    out = pl.pallas_call(
        _mm3_body,
        grid=(GRID,),
        in_specs=[
            pl.BlockSpec((2, 3, BLK, D), lambda i: (0, 0, i, 0)),
            _row_spec(D), _row_spec(D), _row_spec(D), _row_spec(8),
            _full_spec((1, D)),
            _full_spec((3 * D, OUT)),
            _full_spec((1, OUT)),
        ],
        out_specs=_row_spec(OUT),
        out_shape=jax.ShapeDtypeStruct((NP, OUT), f32),
    )(Q, h0, h1, h2, dinv, b2.astype(f32), WcT, bc2)

    return out[:N]


# speedup vs baseline: 12.1222x; 12.1222x over previous
import functools

import jax
import jax.numpy as jnp
from jax import lax
from jax.experimental import pallas as pl
from jax.experimental.pallas import tpu as pltpu
from jax.experimental.pallas import tpu_sc as plsc

N = 10000
NP = 10240
E = 320000
D = 128
NT = 32
EPT = E // NT
CH = 128
NCHUNK = NP // CH
PAD_COL = NP - 1
SCH = 16

def _deg_body(col_hbm, w_hbm, out_hbm, acc, colb, wb, bufa):
    c = lax.axis_index("c")
    s = lax.axis_index("s")
    t = c * 16 + s
    rows_per_tile = NP // 16
    zeros16 = jnp.zeros((16,), jnp.float32)
    ones16 = jnp.ones((16,), jnp.float32)

    for k in range(3):
        def _z(i, _):
            for sub in range(8):
                bufa[i, pl.ds(sub * 16, 16)] = zeros16
            return 0
        lax.fori_loop(0, CH, _z, 0)
        for z in range(rows_per_tile // CH):
            pltpu.sync_copy(bufa, acc.at[pl.ds(s * rows_per_tile + z * CH, CH)])
        plsc.subcore_barrier()

        def _sc_body(sc, _):
            pltpu.sync_copy(col_hbm.at[k, t, pl.ds(sc * SCH, SCH)], colb)
            pltpu.sync_copy(w_hbm.at[k, t, pl.ds(sc * SCH * CH, SCH * CH)], wb)

            def _chunk(q, _):
                def _g(g, _):
                    wvec = wb[pl.ds(q * CH + g * 16, 16)]
                    for i in range(16):
                        bufa[g * 16 + i, pl.ds(0, 16)] = wvec[i] * ones16
                    return 0
                lax.fori_loop(0, CH // 16, _g, 0)
                pltpu.sync_copy(bufa, acc.at[colb.at[q]], add=True)
                return 0

            lax.fori_loop(0, SCH, _chunk, 0)
            return 0

        lax.fori_loop(0, NCHUNK // SCH, _sc_body, 0)
        plsc.subcore_barrier()
        for z in range(rows_per_tile // CH):
            off = s * rows_per_tile + z * CH
            pltpu.sync_copy(acc.at[pl.ds(off, CH)],
                            out_hbm.at[c, k, pl.ds(off, CH)])


def _conv_body(gt_hbm, row_hbm, col_hbm, w_hbm, out_hbm,
               acc, rowb, colb, wb, bufa, bufb, semA, semB):
    c = lax.axis_index("c")
    s = lax.axis_index("s")
    t = c * 16 + s
    rows_per_tile = NP // 16
    zeros16 = jnp.zeros((16,), jnp.float32)

    def _zero_bufa():
        def _z(i, _):
            for sub in range(8):
                bufa[i, pl.ds(sub * 16, 16)] = zeros16
            return 0
        lax.fori_loop(0, CH, _z, 0)

    def _scale(buf, q):
        def _g(g, _):
            wvec = wb[pl.ds(q * CH + g * 16, 16)]
            for i in range(16):
                wv = wvec[i]
                e = g * 16 + i
                for sub in range(8):
                    sl = pl.ds(sub * 16, 16)
                    buf[e, sl] = buf[e, sl] * wv
            return 0

        lax.fori_loop(0, CH // 16, _g, 0)

    for k in range(3):
        _zero_bufa()
        for z in range(rows_per_tile // CH):
            pltpu.sync_copy(bufa, acc.at[pl.ds(s * rows_per_tile + z * CH, CH)])
        plsc.subcore_barrier()

        def _sc_body(sc, _):
            pltpu.sync_copy(row_hbm.at[k, t, pl.ds(sc * SCH, SCH)], rowb)
            pltpu.sync_copy(col_hbm.at[k, t, pl.ds(sc * SCH, SCH)], colb)
            pltpu.sync_copy(w_hbm.at[k, t, pl.ds(sc * SCH * CH, SCH * CH)], wb)

            def _pair(j, _):
                qa = 2 * j
                qb = 2 * j + 1
                cpa = pltpu.async_copy(gt_hbm.at[rowb.at[qa]], bufa, semA)
                cpb = pltpu.async_copy(gt_hbm.at[rowb.at[qb]], bufb, semB)
                cpa.wait()
                _scale(bufa, qa)
                pltpu.sync_copy(bufa, acc.at[colb.at[qa]], add=True)
                cpb.wait()
                _scale(bufb, qb)
                pltpu.sync_copy(bufb, acc.at[colb.at[qb]], add=True)
                return 0

            lax.fori_loop(0, SCH // 2, _pair, 0)
            return 0

        lax.fori_loop(0, NCHUNK // SCH, _sc_body, 0)
        plsc.subcore_barrier()
        for z in range(rows_per_tile // CH):
            off = s * rows_per_tile + z * CH
            pltpu.sync_copy(acc.at[pl.ds(off, CH)],
                            out_hbm.at[c, k, pl.ds(off, CH)])


def _mm1_body(x_ref, w1t_ref, sdeg_ref, g0_ref, g1_ref, g2_ref, dinv_ref):
    h = jnp.dot(x_ref[...], w1t_ref[...])
    gs = (g0_ref, g1_ref, g2_ref)
    dcols = []
    for k in range(3):
        deg = sdeg_ref[0, k, :, 0:1] + sdeg_ref[1, k, :, 0:1] + 1.0
        dinv = jnp.where(deg > 0, lax.rsqrt(deg), 0.0)
        dcols.append(dinv)
        gs[k][...] = h * dinv
    dinv_ref[...] = jnp.concatenate(
        dcols + [jnp.zeros((dcols[0].shape[0], 5), dcols[0].dtype)], axis=1)


def _mm2_body(S_ref, g0_ref, g1_ref, g2_ref, dinv_ref, b_ref, w2t_ref,
              o0_ref, o1_ref, o2_ref):
    dinv = dinv_ref[...]
    b = b_ref[...]
    gs = (g0_ref[...], g1_ref[...], g2_ref[...])
    h2 = None
    for k in range(3):
        a = jnp.maximum(
            dinv[:, k:k + 1] * (S_ref[0, k] + S_ref[1, k] + gs[k]) + b, 0.0)
        p = jnp.dot(a, w2t_ref[pl.ds(k * D, D), :])
        h2 = p if h2 is None else h2 + p
    o0_ref[...] = h2 * dinv[:, 0:1]
    o1_ref[...] = h2 * dinv[:, 1:2]
    o2_ref[...] = h2 * dinv[:, 2:3]


def _mm3_body(Q_ref, g0_ref, g1_ref, g2_ref, dinv_ref, b_ref, wct_ref,
              bc_ref, out_ref):
    dinv = dinv_ref[...]
    b = b_ref[...]
    gs = (g0_ref[...], g1_ref[...], g2_ref[...])
    logits = None
    for k in range(3):
        a = jnp.maximum(
            dinv[:, k:k + 1] * (Q_ref[0, k] + Q_ref[1, k] + gs[k]) + b, 0.0)
        p = jnp.dot(a, wct_ref[pl.ds(k * D, D), :])
        logits = p if logits is None else logits + p
    logits = logits + bc_ref[...]
    m = jnp.max(logits, axis=1, keepdims=True)
    sh = logits - m
    lse = jnp.log(jnp.sum(jnp.exp(sh), axis=1, keepdims=True))
    out_ref[...] = sh - lse


_SC_KERNELS = None


def _get_sc_kernels():
    global _SC_KERNELS
    if _SC_KERNELS is None:
        mesh = plsc.VectorSubcoreMesh(core_axis_name="c", subcore_axis_name="s")
        deg = functools.partial(
            pl.kernel,
            mesh=mesh,
            out_type=jax.ShapeDtypeStruct((2, 3, NP, D), jnp.float32),
            scratch_types=[
                pltpu.VMEM_SHARED((NP, D), jnp.float32),
                pltpu.VMEM((SCH, CH), jnp.int32),
                pltpu.VMEM((SCH * CH,), jnp.float32),
                pltpu.VMEM((CH, D), jnp.float32),
            ],
        )(_deg_body)
        conv = functools.partial(
            pl.kernel,
            mesh=mesh,
            out_type=jax.ShapeDtypeStruct((2, 3, NP, D), jnp.float32),
            scratch_types=[
                pltpu.VMEM_SHARED((NP, D), jnp.float32),
                pltpu.VMEM((SCH, CH), jnp.int32),
                pltpu.VMEM((SCH, CH), jnp.int32),
                pltpu.VMEM((SCH * CH,), jnp.float32),
                pltpu.VMEM((CH, D), jnp.float32),
                pltpu.VMEM((CH, D), jnp.float32),
                pltpu.SemaphoreType.DMA,
                pltpu.SemaphoreType.DMA,
            ],
        )(_conv_body)
        _SC_KERNELS = (deg, conv)
    return _SC_KERNELS


BLK = 1024
GRID = NP // BLK


def _row_spec(width):
    return pl.BlockSpec((BLK, width), lambda i: (i, 0))


def _full_spec(shape):
    return pl.BlockSpec(shape, lambda i: tuple(0 for _ in shape))


def kernel(x, edge_index, edge_in, edge_out, in_w, out_w, W1, b1, W2, b2, Wc, bc):
    f32 = jnp.float32
    i32 = jnp.int32

    ones_w = jnp.ones((E,), f32)
    rows = jnp.stack([edge_index[0], edge_in[0], edge_out[0]]).astype(i32)
    cols = jnp.stack([edge_index[1], edge_in[1], edge_out[1]]).astype(i32)
    ws = jnp.stack([ones_w, in_w.astype(f32), out_w.astype(f32)])
    rows = rows + (jnp.arange(3, dtype=i32) * NP)[:, None]

    pad = NT * NCHUNK * CH - E
    rows_p = jnp.pad(rows.reshape(3, NT, EPT), ((0, 0), (0, 0), (0, pad // NT)),
                     constant_values=0).reshape(3, NT, NCHUNK, CH)
    cols_p = jnp.pad(cols.reshape(3, NT, EPT), ((0, 0), (0, 0), (0, pad // NT)),
                     constant_values=PAD_COL).reshape(3, NT, NCHUNK, CH)
    ws_p = jnp.pad(ws.reshape(3, NT, EPT), ((0, 0), (0, 0), (0, pad // NT)),
                   constant_values=0.0).reshape(3, NT, NCHUNK, CH)

    x_p = jnp.pad(x.astype(f32), ((0, NP - N), (0, 0)))
    W1T = W1.astype(f32).T
    W2T = W2.astype(f32).T
    WcT = Wc.astype(f32).T
    OUT = Wc.shape[0]
    bc2 = bc.astype(f32)[None, :]

    ws_flat = ws_p.reshape(3, NT, NCHUNK * CH)

    _deg_kernel, _conv_kernel = _get_sc_kernels()
    Sdeg = _deg_kernel(cols_p, ws_flat)


    g0, g1, g2, dinv = pl.pallas_call(
        _mm1_body,
        grid=(GRID,),
        in_specs=[
            _row_spec(D),
            _full_spec((D, D)),
            pl.BlockSpec((2, 3, BLK, D), lambda i: (0, 0, i, 0)),
        ],
        out_specs=[_row_spec(D), _row_spec(D), _row_spec(D), _row_spec(8)],
        out_shape=[
            jax.ShapeDtypeStruct((NP, D), f32),
            jax.ShapeDtypeStruct((NP, D), f32),
            jax.ShapeDtypeStruct((NP, D), f32),
            jax.ShapeDtypeStruct((NP, 8), f32),
        ],
    )(x_p, W1T, Sdeg)

    gt = jnp.concatenate([g0, g1, g2], axis=0)

    S = _conv_kernel(gt, rows_p, cols_p, ws_flat)

    h0, h1, h2 = pl.pallas_call(
        _mm2_body,
        grid=(GRID,),
        in_specs=[
            pl.BlockSpec((2, 3, BLK, D), lambda i: (0, 0, i, 0)),
            _row_spec(D), _row_spec(D), _row_spec(D), _row_spec(8),
            _full_spec((1, D)),
            _full_spec((3 * D, D)),
        ],
        out_specs=[_row_spec(D), _row_spec(D), _row_spec(D)],
        out_shape=[
            jax.ShapeDtypeStruct((NP, D), f32),
            jax.ShapeDtypeStruct((NP, D), f32),
            jax.ShapeDtypeStruct((NP, D), f32),
        ],
    )(S, g0, g1, g2, dinv, b1.astype(f32), W2T)

    gt2 = jnp.concatenate([h0, h1, h2], axis=0)

    Q = _conv_kernel(gt2, rows_p, cols_p, ws_flat)

    out = pl.pallas_call(
        _mm3_body,
        grid=(GRID,),
        in_specs=[
            pl.BlockSpec((2, 3, BLK, D), lambda i: (0, 0, i, 0)),
            _row_spec(D), _row_spec(D), _row_spec(D), _row_spec(8),
            _full_spec((1, D)),
            _full_spec((3 * D, OUT)),
            _full_spec((1, OUT)),
        ],
        out_specs=_row_spec(OUT),
        out_shape=jax.ShapeDtypeStruct((NP, OUT), f32),
    )(Q, h0, h1, h2, dinv, b2.astype(f32), WcT, bc2)

    return out[:N]


# speedup vs baseline: 12.7939x; 1.0554x over previous
import functools

import jax
import jax.numpy as jnp
from jax import lax
from jax.experimental import pallas as pl
from jax.experimental.pallas import tpu as pltpu
from jax.experimental.pallas import tpu_sc as plsc

N = 10000
NP = 10240
E = 320000
D = 128
NT = 32
EPT = E // NT
CH = 128
NCHUNK = NP // CH
PAD_COL = NP - 1
SCH = 16

def _deg_body(col_hbm, w_hbm, out_hbm, acc, colb, wb, bufa):
    c = lax.axis_index("c")
    s = lax.axis_index("s")
    t = c * 16 + s
    rows_per_tile = NP // 16
    zeros16 = jnp.zeros((16,), jnp.float32)
    ones16 = jnp.ones((16,), jnp.float32)

    for k in range(3):
        def _z(i, _):
            for sub in range(8):
                bufa[i, pl.ds(sub * 16, 16)] = zeros16
            return 0
        lax.fori_loop(0, CH, _z, 0)
        for z in range(rows_per_tile // CH):
            pltpu.sync_copy(bufa, acc.at[pl.ds(s * rows_per_tile + z * CH, CH)])
        plsc.subcore_barrier()

        def _sc_body(sc, _):
            pltpu.sync_copy(col_hbm.at[k, t, pl.ds(sc * SCH, SCH)], colb)
            pltpu.sync_copy(w_hbm.at[k, t, pl.ds(sc * SCH * CH, SCH * CH)], wb)

            def _chunk(q, _):
                def _g(g, _):
                    wvec = wb[pl.ds(q * CH + g * 16, 16)]
                    for i in range(16):
                        bufa[g * 16 + i, pl.ds(0, 16)] = wvec[i] * ones16
                    return 0
                lax.fori_loop(0, CH // 16, _g, 0)
                pltpu.sync_copy(bufa, acc.at[colb.at[q]], add=True)
                return 0

            lax.fori_loop(0, SCH, _chunk, 0)
            return 0

        lax.fori_loop(0, NCHUNK // SCH, _sc_body, 0)
        plsc.subcore_barrier()
        for z in range(rows_per_tile // CH):
            off = s * rows_per_tile + z * CH
            pltpu.sync_copy(acc.at[pl.ds(off, CH)],
                            out_hbm.at[c, k, pl.ds(off, CH)])


def _conv_body(gt_hbm, row_hbm, col_hbm, w_hbm, out_hbm,
               acc, rowb, colb, wb, bufa, bufb, semA, semB, semSA, semSB):
    c = lax.axis_index("c")
    s = lax.axis_index("s")
    t = c * 16 + s
    rows_per_tile = NP // 16
    zeros16 = jnp.zeros((16,), jnp.float32)

    def _zero_bufa():
        def _z(i, _):
            for sub in range(8):
                bufa[i, pl.ds(sub * 16, 16)] = zeros16
            return 0
        lax.fori_loop(0, CH, _z, 0)

    def _scale(buf, q):
        def _g(g, _):
            wvec = wb[pl.ds(q * CH + g * 16, 16)]
            for i in range(16):
                wv = wvec[i]
                e = g * 16 + i
                for sub in range(8):
                    sl = pl.ds(sub * 16, 16)
                    buf[e, sl] = buf[e, sl] * wv
            return 0

        lax.fori_loop(0, CH // 16, _g, 0)

    for k in range(3):
        _zero_bufa()
        for z in range(rows_per_tile // CH):
            pltpu.sync_copy(bufa, acc.at[pl.ds(s * rows_per_tile + z * CH, CH)])
        plsc.subcore_barrier()

        def _sc_body(sc, _):
            pltpu.sync_copy(row_hbm.at[k, t, pl.ds(sc * SCH, SCH)], rowb)
            pltpu.sync_copy(col_hbm.at[k, t, pl.ds(sc * SCH, SCH)], colb)
            pltpu.sync_copy(w_hbm.at[k, t, pl.ds(sc * SCH * CH, SCH * CH)], wb)

            def _pair(j, _):
                qa = 2 * j
                qb = 2 * j + 1
                cpa = pltpu.async_copy(gt_hbm.at[rowb.at[qa]], bufa, semA)
                cpb = pltpu.async_copy(gt_hbm.at[rowb.at[qb]], bufb, semB)
                cpa.wait()
                if k > 0:
                    _scale(bufa, qa)
                sca = pltpu.async_copy(bufa, acc.at[colb.at[qa]], semSA,
                                       add=True)
                cpb.wait()
                if k > 0:
                    _scale(bufb, qb)
                scb = pltpu.async_copy(bufb, acc.at[colb.at[qb]], semSB,
                                       add=True)
                sca.wait()
                scb.wait()
                return 0

            lax.fori_loop(0, SCH // 2, _pair, 0)
            return 0

        lax.fori_loop(0, NCHUNK // SCH, _sc_body, 0)
        plsc.subcore_barrier()
        for z in range(rows_per_tile // CH):
            off = s * rows_per_tile + z * CH
            pltpu.sync_copy(acc.at[pl.ds(off, CH)],
                            out_hbm.at[c, k, pl.ds(off, CH)])


def _mm1_body(x_ref, w1t_ref, sdeg_ref, g0_ref, g1_ref, g2_ref, dinv_ref):
    h = jnp.dot(x_ref[...], w1t_ref[...])
    gs = (g0_ref, g1_ref, g2_ref)
    dcols = []
    for k in range(3):
        deg = sdeg_ref[0, k, :, 0:1] + sdeg_ref[1, k, :, 0:1] + 1.0
        dinv = jnp.where(deg > 0, lax.rsqrt(deg), 0.0)
        dcols.append(dinv)
        gs[k][...] = h * dinv
    dinv_ref[...] = jnp.concatenate(
        dcols + [jnp.zeros((dcols[0].shape[0], 5), dcols[0].dtype)], axis=1)


def _mm2_body(S_ref, g0_ref, g1_ref, g2_ref, dinv_ref, b_ref, w2t_ref,
              o0_ref, o1_ref, o2_ref):
    dinv = dinv_ref[...]
    b = b_ref[...]
    gs = (g0_ref[...], g1_ref[...], g2_ref[...])
    h2 = None
    for k in range(3):
        a = jnp.maximum(
            dinv[:, k:k + 1] * (S_ref[0, k] + S_ref[1, k] + gs[k]) + b, 0.0)
        p = jnp.dot(a, w2t_ref[pl.ds(k * D, D), :])
        h2 = p if h2 is None else h2 + p
    o0_ref[...] = h2 * dinv[:, 0:1]
    o1_ref[...] = h2 * dinv[:, 1:2]
    o2_ref[...] = h2 * dinv[:, 2:3]


def _mm3_body(Q_ref, g0_ref, g1_ref, g2_ref, dinv_ref, b_ref, wct_ref,
              bc_ref, out_ref):
    dinv = dinv_ref[...]
    b = b_ref[...]
    gs = (g0_ref[...], g1_ref[...], g2_ref[...])
    logits = None
    for k in range(3):
        a = jnp.maximum(
            dinv[:, k:k + 1] * (Q_ref[0, k] + Q_ref[1, k] + gs[k]) + b, 0.0)
        p = jnp.dot(a, wct_ref[pl.ds(k * D, D), :])
        logits = p if logits is None else logits + p
    logits = logits + bc_ref[...]
    m = jnp.max(logits, axis=1, keepdims=True)
    sh = logits - m
    lse = jnp.log(jnp.sum(jnp.exp(sh), axis=1, keepdims=True))
    out_ref[...] = sh - lse


_SC_KERNELS = None


def _get_sc_kernels():
    global _SC_KERNELS
    if _SC_KERNELS is None:
        mesh = plsc.VectorSubcoreMesh(core_axis_name="c", subcore_axis_name="s")
        deg = functools.partial(
            pl.kernel,
            mesh=mesh,
            out_type=jax.ShapeDtypeStruct((2, 3, NP, D), jnp.float32),
            scratch_types=[
                pltpu.VMEM_SHARED((NP, D), jnp.float32),
                pltpu.VMEM((SCH, CH), jnp.int32),
                pltpu.VMEM((SCH * CH,), jnp.float32),
                pltpu.VMEM((CH, D), jnp.float32),
            ],
        )(_deg_body)
        conv = functools.partial(
            pl.kernel,
            mesh=mesh,
            out_type=jax.ShapeDtypeStruct((2, 3, NP, D), jnp.float32),
            scratch_types=[
                pltpu.VMEM_SHARED((NP, D), jnp.float32),
                pltpu.VMEM((SCH, CH), jnp.int32),
                pltpu.VMEM((SCH, CH), jnp.int32),
                pltpu.VMEM((SCH * CH,), jnp.float32),
                pltpu.VMEM((CH, D), jnp.float32),
                pltpu.VMEM((CH, D), jnp.float32),
                pltpu.SemaphoreType.DMA,
                pltpu.SemaphoreType.DMA,
                pltpu.SemaphoreType.DMA,
                pltpu.SemaphoreType.DMA,
            ],
        )(_conv_body)
        _SC_KERNELS = (deg, conv)
    return _SC_KERNELS


BLK = 1024
GRID = NP // BLK


def _row_spec(width):
    return pl.BlockSpec((BLK, width), lambda i: (i, 0))


def _full_spec(shape):
    return pl.BlockSpec(shape, lambda i: tuple(0 for _ in shape))


def kernel(x, edge_index, edge_in, edge_out, in_w, out_w, W1, b1, W2, b2, Wc, bc):
    f32 = jnp.float32
    i32 = jnp.int32

    ones_w = jnp.ones((E,), f32)
    rows = jnp.stack([edge_index[0], edge_in[0], edge_out[0]]).astype(i32)
    cols = jnp.stack([edge_index[1], edge_in[1], edge_out[1]]).astype(i32)
    ws = jnp.stack([ones_w, in_w.astype(f32), out_w.astype(f32)])
    rows = rows + (jnp.arange(3, dtype=i32) * NP)[:, None]

    pad = NT * NCHUNK * CH - E
    rows_p = jnp.pad(rows.reshape(3, NT, EPT), ((0, 0), (0, 0), (0, pad // NT)),
                     constant_values=0).reshape(3, NT, NCHUNK, CH)
    cols_p = jnp.pad(cols.reshape(3, NT, EPT), ((0, 0), (0, 0), (0, pad // NT)),
                     constant_values=PAD_COL).reshape(3, NT, NCHUNK, CH)
    ws_p = jnp.pad(ws.reshape(3, NT, EPT), ((0, 0), (0, 0), (0, pad // NT)),
                   constant_values=0.0).reshape(3, NT, NCHUNK, CH)

    x_p = jnp.pad(x.astype(f32), ((0, NP - N), (0, 0)))
    W1T = W1.astype(f32).T
    W2T = W2.astype(f32).T
    WcT = Wc.astype(f32).T
    OUT = Wc.shape[0]
    bc2 = bc.astype(f32)[None, :]

    ws_flat = ws_p.reshape(3, NT, NCHUNK * CH)

    _deg_kernel, _conv_kernel = _get_sc_kernels()
    Sdeg = _deg_kernel(cols_p, ws_flat)


    g0, g1, g2, dinv = pl.pallas_call(
        _mm1_body,
        grid=(GRID,),
        in_specs=[
            _row_spec(D),
            _full_spec((D, D)),
            pl.BlockSpec((2, 3, BLK, D), lambda i: (0, 0, i, 0)),
        ],
        out_specs=[_row_spec(D), _row_spec(D), _row_spec(D), _row_spec(8)],
        out_shape=[
            jax.ShapeDtypeStruct((NP, D), f32),
            jax.ShapeDtypeStruct((NP, D), f32),
            jax.ShapeDtypeStruct((NP, D), f32),
            jax.ShapeDtypeStruct((NP, 8), f32),
        ],
    )(x_p, W1T, Sdeg)

    gt = jnp.concatenate([g0, g1, g2], axis=0)

    S = _conv_kernel(gt, rows_p, cols_p, ws_flat)

    h0, h1, h2 = pl.pallas_call(
        _mm2_body,
        grid=(GRID,),
        in_specs=[
            pl.BlockSpec((2, 3, BLK, D), lambda i: (0, 0, i, 0)),
            _row_spec(D), _row_spec(D), _row_spec(D), _row_spec(8),
            _full_spec((1, D)),
            _full_spec((3 * D, D)),
        ],
        out_specs=[_row_spec(D), _row_spec(D), _row_spec(D)],
        out_shape=[
            jax.ShapeDtypeStruct((NP, D), f32),
            jax.ShapeDtypeStruct((NP, D), f32),
            jax.ShapeDtypeStruct((NP, D), f32),
        ],
    )(S, g0, g1, g2, dinv, b1.astype(f32), W2T)

    gt2 = jnp.concatenate([h0, h1, h2], axis=0)

    Q = _conv_kernel(gt2, rows_p, cols_p, ws_flat)

    out = pl.pallas_call(
        _mm3_body,
        grid=(GRID,),
        in_specs=[
            pl.BlockSpec((2, 3, BLK, D), lambda i: (0, 0, i, 0)),
            _row_spec(D), _row_spec(D), _row_spec(D), _row_spec(8),
            _full_spec((1, D)),
            _full_spec((3 * D, OUT)),
            _full_spec((1, OUT)),
        ],
        out_specs=_row_spec(OUT),
        out_shape=jax.ShapeDtypeStruct((NP, OUT), f32),
    )(Q, h0, h1, h2, dinv, b2.astype(f32), WcT, bc2)

    return out[:N]


# speedup vs baseline: 13.7266x; 1.0729x over previous
import functools

import jax
import jax.numpy as jnp
from jax import lax
from jax.experimental import pallas as pl
from jax.experimental.pallas import tpu as pltpu
from jax.experimental.pallas import tpu_sc as plsc

N = 10000
NP = 10240
E = 320000
D = 128
NT = 32
EPT = E // NT
CH = 128
NCHUNK = NP // CH
PAD_COL = NP - 1
SCH = 16

def _deg_body(col_hbm, w_hbm, out_hbm, acc, colb, wb, bufa):
    c = lax.axis_index("c")
    s = lax.axis_index("s")
    t = c * 16 + s
    rows_per_tile = NP // 16
    zeros16 = jnp.zeros((16,), jnp.float32)
    ones16 = jnp.ones((16,), jnp.float32)

    for k in range(3):
        def _z(i, _):
            for sub in range(8):
                bufa[i, pl.ds(sub * 16, 16)] = zeros16
            return 0
        lax.fori_loop(0, CH, _z, 0)
        for z in range(rows_per_tile // CH):
            pltpu.sync_copy(bufa, acc.at[pl.ds(s * rows_per_tile + z * CH, CH)])
        plsc.subcore_barrier()

        def _sc_body(sc, _):
            pltpu.sync_copy(col_hbm.at[k, t, pl.ds(sc * SCH, SCH)], colb)
            pltpu.sync_copy(w_hbm.at[k, t, pl.ds(sc * SCH * CH, SCH * CH)], wb)

            def _chunk(q, _):
                def _g(g, _):
                    wvec = wb[pl.ds(q * CH + g * 16, 16)]
                    for i in range(16):
                        bufa[g * 16 + i, pl.ds(0, 16)] = wvec[i] * ones16
                    return 0
                lax.fori_loop(0, CH // 16, _g, 0)
                pltpu.sync_copy(bufa, acc.at[colb.at[q]], add=True)
                return 0

            lax.fori_loop(0, SCH, _chunk, 0)
            return 0

        lax.fori_loop(0, NCHUNK // SCH, _sc_body, 0)
        plsc.subcore_barrier()
        for z in range(rows_per_tile // CH):
            off = s * rows_per_tile + z * CH
            pltpu.sync_copy(acc.at[pl.ds(off, CH)],
                            out_hbm.at[c, k, pl.ds(off, CH)])


def _conv_body(gt_hbm, row_hbm, col_hbm, w_hbm, out_hbm,
               acc, rowb, colb, wb, bufa, bufb, semA, semB, semSA, semSB):
    c = lax.axis_index("c")
    s = lax.axis_index("s")
    t = c * 16 + s
    rows_per_tile = NP // 16
    zeros16 = jnp.zeros((16,), jnp.float32)

    def _zero_bufa():
        def _z(i, _):
            for sub in range(8):
                bufa[i, pl.ds(sub * 16, 16)] = zeros16
            return 0
        lax.fori_loop(0, CH, _z, 0)

    def _scale(buf, q):
        def _g(g, _):
            wvec = wb[pl.ds(q * CH + g * 16, 16)]
            for i in range(16):
                wv = wvec[i]
                e = g * 16 + i
                for sub in range(8):
                    sl = pl.ds(sub * 16, 16)
                    buf[e, sl] = buf[e, sl] * wv
            return 0

        lax.fori_loop(0, CH // 16, _g, 0)

    for k in range(3):
        _zero_bufa()
        for z in range(rows_per_tile // CH):
            pltpu.sync_copy(bufa, acc.at[pl.ds(s * rows_per_tile + z * CH, CH)])
        plsc.subcore_barrier()

        def _sc_body(sc, _):
            pltpu.sync_copy(row_hbm.at[k, t, pl.ds(sc * SCH, SCH)], rowb)
            pltpu.sync_copy(col_hbm.at[k, t, pl.ds(sc * SCH, SCH)], colb)
            pltpu.sync_copy(w_hbm.at[k, t, pl.ds(sc * SCH * CH, SCH * CH)], wb)

            pltpu.async_copy(gt_hbm.at[rowb.at[0]], bufa, semA)
            pltpu.async_copy(gt_hbm.at[rowb.at[1]], bufb, semB)

            def _pair(j, _):
                qa = 2 * j
                qb = 2 * j + 1
                pltpu.make_async_copy(gt_hbm.at[rowb.at[qa]], bufa,
                                      semA).wait()
                if k > 0:
                    _scale(bufa, qa)
                sa = pltpu.async_copy(bufa, acc.at[colb.at[qa]], semSA,
                                      add=True)
                pltpu.make_async_copy(gt_hbm.at[rowb.at[qb]], bufb,
                                      semB).wait()
                sa.wait()
                pltpu.async_copy(gt_hbm.at[rowb.at[qa + 2]], bufa, semA)
                if k > 0:
                    _scale(bufb, qb)
                sb = pltpu.async_copy(bufb, acc.at[colb.at[qb]], semSB,
                                      add=True)
                sb.wait()
                pltpu.async_copy(gt_hbm.at[rowb.at[qb + 2]], bufb, semB)
                return 0

            lax.fori_loop(0, SCH // 2 - 1, _pair, 0)

            qa = SCH - 2
            qb = SCH - 1
            pltpu.make_async_copy(gt_hbm.at[rowb.at[qa]], bufa, semA).wait()
            if k > 0:
                _scale(bufa, qa)
            sa = pltpu.async_copy(bufa, acc.at[colb.at[qa]], semSA, add=True)
            pltpu.make_async_copy(gt_hbm.at[rowb.at[qb]], bufb, semB).wait()
            sa.wait()
            if k > 0:
                _scale(bufb, qb)
            sb = pltpu.async_copy(bufb, acc.at[colb.at[qb]], semSB, add=True)
            sb.wait()
            return 0

        lax.fori_loop(0, NCHUNK // SCH, _sc_body, 0)
        plsc.subcore_barrier()
        for z in range(rows_per_tile // CH):
            off = s * rows_per_tile + z * CH
            pltpu.sync_copy(acc.at[pl.ds(off, CH)],
                            out_hbm.at[c, k, pl.ds(off, CH)])


def _mm1_body(x_ref, w1t_ref, sdeg_ref, g0_ref, g1_ref, g2_ref, dinv_ref):
    h = jnp.dot(x_ref[...], w1t_ref[...])
    gs = (g0_ref, g1_ref, g2_ref)
    dcols = []
    for k in range(3):
        deg = sdeg_ref[0, k, :, 0:1] + sdeg_ref[1, k, :, 0:1] + 1.0
        dinv = jnp.where(deg > 0, lax.rsqrt(deg), 0.0)
        dcols.append(dinv)
        gs[k][...] = h * dinv
    dinv_ref[...] = jnp.concatenate(
        dcols + [jnp.zeros((dcols[0].shape[0], 5), dcols[0].dtype)], axis=1)


def _mm2_body(S_ref, g0_ref, g1_ref, g2_ref, dinv_ref, b_ref, w2t_ref,
              o0_ref, o1_ref, o2_ref):
    dinv = dinv_ref[...]
    b = b_ref[...]
    gs = (g0_ref[...], g1_ref[...], g2_ref[...])
    h2 = None
    for k in range(3):
        a = jnp.maximum(
            dinv[:, k:k + 1] * (S_ref[0, k] + S_ref[1, k] + gs[k]) + b, 0.0)
        p = jnp.dot(a, w2t_ref[pl.ds(k * D, D), :])
        h2 = p if h2 is None else h2 + p
    o0_ref[...] = h2 * dinv[:, 0:1]
    o1_ref[...] = h2 * dinv[:, 1:2]
    o2_ref[...] = h2 * dinv[:, 2:3]


def _mm3_body(Q_ref, g0_ref, g1_ref, g2_ref, dinv_ref, b_ref, wct_ref,
              bc_ref, out_ref):
    dinv = dinv_ref[...]
    b = b_ref[...]
    gs = (g0_ref[...], g1_ref[...], g2_ref[...])
    logits = None
    for k in range(3):
        a = jnp.maximum(
            dinv[:, k:k + 1] * (Q_ref[0, k] + Q_ref[1, k] + gs[k]) + b, 0.0)
        p = jnp.dot(a, wct_ref[pl.ds(k * D, D), :])
        logits = p if logits is None else logits + p
    logits = logits + bc_ref[...]
    m = jnp.max(logits, axis=1, keepdims=True)
    sh = logits - m
    lse = jnp.log(jnp.sum(jnp.exp(sh), axis=1, keepdims=True))
    out_ref[...] = sh - lse


_SC_KERNELS = None


def _get_sc_kernels():
    global _SC_KERNELS
    if _SC_KERNELS is None:
        mesh = plsc.VectorSubcoreMesh(core_axis_name="c", subcore_axis_name="s")
        deg = functools.partial(
            pl.kernel,
            mesh=mesh,
            out_type=jax.ShapeDtypeStruct((2, 3, NP, D), jnp.float32),
            scratch_types=[
                pltpu.VMEM_SHARED((NP, D), jnp.float32),
                pltpu.VMEM((SCH, CH), jnp.int32),
                pltpu.VMEM((SCH * CH,), jnp.float32),
                pltpu.VMEM((CH, D), jnp.float32),
            ],
        )(_deg_body)
        conv = functools.partial(
            pl.kernel,
            mesh=mesh,
            out_type=jax.ShapeDtypeStruct((2, 3, NP, D), jnp.float32),
            scratch_types=[
                pltpu.VMEM_SHARED((NP, D), jnp.float32),
                pltpu.VMEM((SCH, CH), jnp.int32),
                pltpu.VMEM((SCH, CH), jnp.int32),
                pltpu.VMEM((SCH * CH,), jnp.float32),
                pltpu.VMEM((CH, D), jnp.float32),
                pltpu.VMEM((CH, D), jnp.float32),
                pltpu.SemaphoreType.DMA,
                pltpu.SemaphoreType.DMA,
                pltpu.SemaphoreType.DMA,
                pltpu.SemaphoreType.DMA,
            ],
        )(_conv_body)
        _SC_KERNELS = (deg, conv)
    return _SC_KERNELS


BLK = 1024
GRID = NP // BLK


def _row_spec(width):
    return pl.BlockSpec((BLK, width), lambda i: (i, 0))


def _full_spec(shape):
    return pl.BlockSpec(shape, lambda i: tuple(0 for _ in shape))


def kernel(x, edge_index, edge_in, edge_out, in_w, out_w, W1, b1, W2, b2, Wc, bc):
    f32 = jnp.float32
    i32 = jnp.int32

    ones_w = jnp.ones((E,), f32)
    rows = jnp.stack([edge_index[0], edge_in[0], edge_out[0]]).astype(i32)
    cols = jnp.stack([edge_index[1], edge_in[1], edge_out[1]]).astype(i32)
    ws = jnp.stack([ones_w, in_w.astype(f32), out_w.astype(f32)])
    rows = rows + (jnp.arange(3, dtype=i32) * NP)[:, None]

    pad = NT * NCHUNK * CH - E
    rows_p = jnp.pad(rows.reshape(3, NT, EPT), ((0, 0), (0, 0), (0, pad // NT)),
                     constant_values=0).reshape(3, NT, NCHUNK, CH)
    cols_p = jnp.pad(cols.reshape(3, NT, EPT), ((0, 0), (0, 0), (0, pad // NT)),
                     constant_values=PAD_COL).reshape(3, NT, NCHUNK, CH)
    ws_p = jnp.pad(ws.reshape(3, NT, EPT), ((0, 0), (0, 0), (0, pad // NT)),
                   constant_values=0.0).reshape(3, NT, NCHUNK, CH)

    x_p = jnp.pad(x.astype(f32), ((0, NP - N), (0, 0)))
    W1T = W1.astype(f32).T
    W2T = W2.astype(f32).T
    WcT = Wc.astype(f32).T
    OUT = Wc.shape[0]
    bc2 = bc.astype(f32)[None, :]

    ws_flat = ws_p.reshape(3, NT, NCHUNK * CH)

    _deg_kernel, _conv_kernel = _get_sc_kernels()
    Sdeg = _deg_kernel(cols_p, ws_flat)


    g0, g1, g2, dinv = pl.pallas_call(
        _mm1_body,
        grid=(GRID,),
        in_specs=[
            _row_spec(D),
            _full_spec((D, D)),
            pl.BlockSpec((2, 3, BLK, D), lambda i: (0, 0, i, 0)),
        ],
        out_specs=[_row_spec(D), _row_spec(D), _row_spec(D), _row_spec(8)],
        out_shape=[
            jax.ShapeDtypeStruct((NP, D), f32),
            jax.ShapeDtypeStruct((NP, D), f32),
            jax.ShapeDtypeStruct((NP, D), f32),
            jax.ShapeDtypeStruct((NP, 8), f32),
        ],
    )(x_p, W1T, Sdeg)

    gt = jnp.concatenate([g0, g1, g2], axis=0)

    S = _conv_kernel(gt, rows_p, cols_p, ws_flat)

    h0, h1, h2 = pl.pallas_call(
        _mm2_body,
        grid=(GRID,),
        in_specs=[
            pl.BlockSpec((2, 3, BLK, D), lambda i: (0, 0, i, 0)),
            _row_spec(D), _row_spec(D), _row_spec(D), _row_spec(8),
            _full_spec((1, D)),
            _full_spec((3 * D, D)),
        ],
        out_specs=[_row_spec(D), _row_spec(D), _row_spec(D)],
        out_shape=[
            jax.ShapeDtypeStruct((NP, D), f32),
            jax.ShapeDtypeStruct((NP, D), f32),
            jax.ShapeDtypeStruct((NP, D), f32),
        ],
    )(S, g0, g1, g2, dinv, b1.astype(f32), W2T)

    gt2 = jnp.concatenate([h0, h1, h2], axis=0)

    Q = _conv_kernel(gt2, rows_p, cols_p, ws_flat)

    out = pl.pallas_call(
        _mm3_body,
        grid=(GRID,),
        in_specs=[
            pl.BlockSpec((2, 3, BLK, D), lambda i: (0, 0, i, 0)),
            _row_spec(D), _row_spec(D), _row_spec(D), _row_spec(8),
            _full_spec((1, D)),
            _full_spec((3 * D, OUT)),
            _full_spec((1, OUT)),
        ],
        out_specs=_row_spec(OUT),
        out_shape=jax.ShapeDtypeStruct((NP, OUT), f32),
    )(Q, h0, h1, h2, dinv, b2.astype(f32), WcT, bc2)

    return out[:N]


# speedup vs baseline: 14.0248x; 1.0217x over previous
import functools

import jax
import jax.numpy as jnp
from jax import lax
from jax.experimental import pallas as pl
from jax.experimental.pallas import tpu as pltpu
from jax.experimental.pallas import tpu_sc as plsc

N = 10000
NP = 10240
E = 320000
D = 128
NT = 32
EPT = E // NT
CH = 128
NCHUNK = NP // CH
PAD_COL = NP - 1
SCH = 40

def _deg_body(col_hbm, w_hbm, out_hbm, acc, colb, wb, bufa):
    c = lax.axis_index("c")
    s = lax.axis_index("s")
    t = c * 16 + s
    rows_per_tile = NP // 16
    zeros16 = jnp.zeros((16,), jnp.float32)
    ones16 = jnp.ones((16,), jnp.float32)

    for k in range(3):
        def _z(i, _):
            for sub in range(8):
                bufa[i, pl.ds(sub * 16, 16)] = zeros16
            return 0
        lax.fori_loop(0, CH, _z, 0)
        for z in range(rows_per_tile // CH):
            pltpu.sync_copy(bufa, acc.at[pl.ds(s * rows_per_tile + z * CH, CH)])
        plsc.subcore_barrier()

        def _sc_body(sc, _):
            pltpu.sync_copy(col_hbm.at[k, t, pl.ds(sc * SCH, SCH)], colb)
            pltpu.sync_copy(w_hbm.at[k, t, pl.ds(sc * SCH * CH, SCH * CH)], wb)

            def _chunk(q, _):
                def _g(g, _):
                    wvec = wb[pl.ds(q * CH + g * 16, 16)]
                    for i in range(16):
                        bufa[g * 16 + i, pl.ds(0, 16)] = wvec[i] * ones16
                    return 0
                lax.fori_loop(0, CH // 16, _g, 0)
                pltpu.sync_copy(bufa, acc.at[colb.at[q]], add=True)
                return 0

            lax.fori_loop(0, SCH, _chunk, 0)
            return 0

        lax.fori_loop(0, NCHUNK // SCH, _sc_body, 0)
        plsc.subcore_barrier()
        for z in range(rows_per_tile // CH):
            off = s * rows_per_tile + z * CH
            pltpu.sync_copy(acc.at[pl.ds(off, CH)],
                            out_hbm.at[c, k, pl.ds(off, CH)])


def _conv_body(gt_hbm, row_hbm, col_hbm, w_hbm, out_hbm,
               acc, rowb, colb, wb, bufa, bufb, semA, semB, semSA, semSB):
    c = lax.axis_index("c")
    s = lax.axis_index("s")
    t = c * 16 + s
    rows_per_tile = NP // 16
    zeros16 = jnp.zeros((16,), jnp.float32)

    def _zero_bufa():
        def _z(i, _):
            for sub in range(8):
                bufa[i, pl.ds(sub * 16, 16)] = zeros16
            return 0
        lax.fori_loop(0, CH, _z, 0)

    def _scale(buf, q):
        def _g(g, _):
            wvec = wb[pl.ds(q * CH + g * 16, 16)]
            for i in range(16):
                wv = wvec[i]
                e = g * 16 + i
                for sub in range(8):
                    sl = pl.ds(sub * 16, 16)
                    buf[e, sl] = buf[e, sl] * wv
            return 0

        lax.fori_loop(0, CH // 16, _g, 0)

    for k in range(3):
        _zero_bufa()
        for z in range(rows_per_tile // CH):
            pltpu.sync_copy(bufa, acc.at[pl.ds(s * rows_per_tile + z * CH, CH)])
        plsc.subcore_barrier()

        def _sc_body(sc, _):
            pltpu.sync_copy(row_hbm.at[k, t, pl.ds(sc * SCH, SCH)], rowb)
            pltpu.sync_copy(col_hbm.at[k, t, pl.ds(sc * SCH, SCH)], colb)
            pltpu.sync_copy(w_hbm.at[k, t, pl.ds(sc * SCH * CH, SCH * CH)], wb)

            pltpu.async_copy(gt_hbm.at[rowb.at[0]], bufa, semA)
            pltpu.async_copy(gt_hbm.at[rowb.at[1]], bufb, semB)

            def _pair(j, _):
                qa = 2 * j
                qb = 2 * j + 1
                pltpu.make_async_copy(gt_hbm.at[rowb.at[qa]], bufa,
                                      semA).wait()
                if k > 0:
                    _scale(bufa, qa)
                sa = pltpu.async_copy(bufa, acc.at[colb.at[qa]], semSA,
                                      add=True)
                pltpu.make_async_copy(gt_hbm.at[rowb.at[qb]], bufb,
                                      semB).wait()
                sa.wait()
                pltpu.async_copy(gt_hbm.at[rowb.at[qa + 2]], bufa, semA)
                if k > 0:
                    _scale(bufb, qb)
                sb = pltpu.async_copy(bufb, acc.at[colb.at[qb]], semSB,
                                      add=True)
                sb.wait()
                pltpu.async_copy(gt_hbm.at[rowb.at[qb + 2]], bufb, semB)
                return 0

            lax.fori_loop(0, SCH // 2 - 1, _pair, 0)

            qa = SCH - 2
            qb = SCH - 1
            pltpu.make_async_copy(gt_hbm.at[rowb.at[qa]], bufa, semA).wait()
            if k > 0:
                _scale(bufa, qa)
            sa = pltpu.async_copy(bufa, acc.at[colb.at[qa]], semSA, add=True)
            pltpu.make_async_copy(gt_hbm.at[rowb.at[qb]], bufb, semB).wait()
            sa.wait()
            if k > 0:
                _scale(bufb, qb)
            sb = pltpu.async_copy(bufb, acc.at[colb.at[qb]], semSB, add=True)
            sb.wait()
            return 0

        lax.fori_loop(0, NCHUNK // SCH, _sc_body, 0)
        plsc.subcore_barrier()
        for z in range(rows_per_tile // CH):
            off = s * rows_per_tile + z * CH
            pltpu.sync_copy(acc.at[pl.ds(off, CH)],
                            out_hbm.at[c, k, pl.ds(off, CH)])


def _mm1_body(x_ref, w1t_ref, sdeg_ref, g0_ref, g1_ref, g2_ref, dinv_ref):
    h = jnp.dot(x_ref[...], w1t_ref[...])
    gs = (g0_ref, g1_ref, g2_ref)
    dcols = []
    for k in range(3):
        deg = sdeg_ref[0, k, :, 0:1] + sdeg_ref[1, k, :, 0:1] + 1.0
        dinv = jnp.where(deg > 0, lax.rsqrt(deg), 0.0)
        dcols.append(dinv)
        gs[k][...] = h * dinv
    dinv_ref[...] = jnp.concatenate(
        dcols + [jnp.zeros((dcols[0].shape[0], 5), dcols[0].dtype)], axis=1)


def _mm2_body(S_ref, g0_ref, g1_ref, g2_ref, dinv_ref, b_ref, w2t_ref,
              o0_ref, o1_ref, o2_ref):
    dinv = dinv_ref[...]
    b = b_ref[...]
    gs = (g0_ref[...], g1_ref[...], g2_ref[...])
    h2 = None
    for k in range(3):
        a = jnp.maximum(
            dinv[:, k:k + 1] * (S_ref[0, k] + S_ref[1, k] + gs[k]) + b, 0.0)
        p = jnp.dot(a, w2t_ref[pl.ds(k * D, D), :])
        h2 = p if h2 is None else h2 + p
    o0_ref[...] = h2 * dinv[:, 0:1]
    o1_ref[...] = h2 * dinv[:, 1:2]
    o2_ref[...] = h2 * dinv[:, 2:3]


def _mm3_body(Q_ref, g0_ref, g1_ref, g2_ref, dinv_ref, b_ref, wct_ref,
              bc_ref, out_ref):
    dinv = dinv_ref[...]
    b = b_ref[...]
    gs = (g0_ref[...], g1_ref[...], g2_ref[...])
    logits = None
    for k in range(3):
        a = jnp.maximum(
            dinv[:, k:k + 1] * (Q_ref[0, k] + Q_ref[1, k] + gs[k]) + b, 0.0)
        p = jnp.dot(a, wct_ref[pl.ds(k * D, D), :])
        logits = p if logits is None else logits + p
    logits = logits + bc_ref[...]
    m = jnp.max(logits, axis=1, keepdims=True)
    sh = logits - m
    lse = jnp.log(jnp.sum(jnp.exp(sh), axis=1, keepdims=True))
    out_ref[...] = sh - lse


_SC_KERNELS = None


def _get_sc_kernels():
    global _SC_KERNELS
    if _SC_KERNELS is None:
        mesh = plsc.VectorSubcoreMesh(core_axis_name="c", subcore_axis_name="s")
        deg = functools.partial(
            pl.kernel,
            mesh=mesh,
            out_type=jax.ShapeDtypeStruct((2, 3, NP, D), jnp.float32),
            scratch_types=[
                pltpu.VMEM_SHARED((NP, D), jnp.float32),
                pltpu.VMEM((SCH, CH), jnp.int32),
                pltpu.VMEM((SCH * CH,), jnp.float32),
                pltpu.VMEM((CH, D), jnp.float32),
            ],
        )(_deg_body)
        conv = functools.partial(
            pl.kernel,
            mesh=mesh,
            out_type=jax.ShapeDtypeStruct((2, 3, NP, D), jnp.float32),
            scratch_types=[
                pltpu.VMEM_SHARED((NP, D), jnp.float32),
                pltpu.VMEM((SCH, CH), jnp.int32),
                pltpu.VMEM((SCH, CH), jnp.int32),
                pltpu.VMEM((SCH * CH,), jnp.float32),
                pltpu.VMEM((CH, D), jnp.float32),
                pltpu.VMEM((CH, D), jnp.float32),
                pltpu.SemaphoreType.DMA,
                pltpu.SemaphoreType.DMA,
                pltpu.SemaphoreType.DMA,
                pltpu.SemaphoreType.DMA,
            ],
        )(_conv_body)
        _SC_KERNELS = (deg, conv)
    return _SC_KERNELS


BLK = 1024
GRID = NP // BLK


def _row_spec(width):
    return pl.BlockSpec((BLK, width), lambda i: (i, 0))


def _full_spec(shape):
    return pl.BlockSpec(shape, lambda i: tuple(0 for _ in shape))


def kernel(x, edge_index, edge_in, edge_out, in_w, out_w, W1, b1, W2, b2, Wc, bc):
    f32 = jnp.float32
    i32 = jnp.int32

    ones_w = jnp.ones((E,), f32)
    rows = jnp.stack([edge_index[0], edge_in[0], edge_out[0]]).astype(i32)
    cols = jnp.stack([edge_index[1], edge_in[1], edge_out[1]]).astype(i32)
    ws = jnp.stack([ones_w, in_w.astype(f32), out_w.astype(f32)])
    rows = rows + (jnp.arange(3, dtype=i32) * NP)[:, None]

    pad = NT * NCHUNK * CH - E
    rows_p = jnp.pad(rows.reshape(3, NT, EPT), ((0, 0), (0, 0), (0, pad // NT)),
                     constant_values=0).reshape(3, NT, NCHUNK, CH)
    cols_p = jnp.pad(cols.reshape(3, NT, EPT), ((0, 0), (0, 0), (0, pad // NT)),
                     constant_values=PAD_COL).reshape(3, NT, NCHUNK, CH)
    ws_p = jnp.pad(ws.reshape(3, NT, EPT), ((0, 0), (0, 0), (0, pad // NT)),
                   constant_values=0.0).reshape(3, NT, NCHUNK, CH)

    x_p = jnp.pad(x.astype(f32), ((0, NP - N), (0, 0)))
    W1T = W1.astype(f32).T
    W2T = W2.astype(f32).T
    WcT = Wc.astype(f32).T
    OUT = Wc.shape[0]
    bc2 = bc.astype(f32)[None, :]

    ws_flat = ws_p.reshape(3, NT, NCHUNK * CH)

    _deg_kernel, _conv_kernel = _get_sc_kernels()
    Sdeg = _deg_kernel(cols_p, ws_flat)


    g0, g1, g2, dinv = pl.pallas_call(
        _mm1_body,
        grid=(GRID,),
        in_specs=[
            _row_spec(D),
            _full_spec((D, D)),
            pl.BlockSpec((2, 3, BLK, D), lambda i: (0, 0, i, 0)),
        ],
        out_specs=[_row_spec(D), _row_spec(D), _row_spec(D), _row_spec(8)],
        out_shape=[
            jax.ShapeDtypeStruct((NP, D), f32),
            jax.ShapeDtypeStruct((NP, D), f32),
            jax.ShapeDtypeStruct((NP, D), f32),
            jax.ShapeDtypeStruct((NP, 8), f32),
        ],
    )(x_p, W1T, Sdeg)

    gt = jnp.concatenate([g0, g1, g2], axis=0)

    S = _conv_kernel(gt, rows_p, cols_p, ws_flat)

    h0, h1, h2 = pl.pallas_call(
        _mm2_body,
        grid=(GRID,),
        in_specs=[
            pl.BlockSpec((2, 3, BLK, D), lambda i: (0, 0, i, 0)),
            _row_spec(D), _row_spec(D), _row_spec(D), _row_spec(8),
            _full_spec((1, D)),
            _full_spec((3 * D, D)),
        ],
        out_specs=[_row_spec(D), _row_spec(D), _row_spec(D)],
        out_shape=[
            jax.ShapeDtypeStruct((NP, D), f32),
            jax.ShapeDtypeStruct((NP, D), f32),
            jax.ShapeDtypeStruct((NP, D), f32),
        ],
    )(S, g0, g1, g2, dinv, b1.astype(f32), W2T)

    gt2 = jnp.concatenate([h0, h1, h2], axis=0)

    Q = _conv_kernel(gt2, rows_p, cols_p, ws_flat)

    out = pl.pallas_call(
        _mm3_body,
        grid=(GRID,),
        in_specs=[
            pl.BlockSpec((2, 3, BLK, D), lambda i: (0, 0, i, 0)),
            _row_spec(D), _row_spec(D), _row_spec(D), _row_spec(8),
            _full_spec((1, D)),
            _full_spec((3 * D, OUT)),
            _full_spec((1, OUT)),
        ],
        out_specs=_row_spec(OUT),
        out_shape=jax.ShapeDtypeStruct((NP, OUT), f32),
    )(Q, h0, h1, h2, dinv, b2.astype(f32), WcT, bc2)

    return out[:N]


# speedup vs baseline: 14.0293x; 1.0003x over previous
import functools

import jax
import jax.numpy as jnp
from jax import lax
from jax.experimental import pallas as pl
from jax.experimental.pallas import tpu as pltpu
from jax.experimental.pallas import tpu_sc as plsc

N = 10000
NP = 10240
E = 320000
D = 128
NT = 32
EPT = E // NT
CH = 128
NCHUNK = NP // CH
PAD_COL = NP - 1
SCH = 40

def _deg_body(col_hbm, w_hbm, out_hbm, acc, colb, wb, bufa):
    c = lax.axis_index("c")
    s = lax.axis_index("s")
    t = c * 16 + s
    rows_per_tile = NP // 16
    zeros16 = jnp.zeros((16,), jnp.float32)
    ones16 = jnp.ones((16,), jnp.float32)

    for k in range(3):
        def _z(i, _):
            for sub in range(8):
                bufa[i, pl.ds(sub * 16, 16)] = zeros16
            return 0
        lax.fori_loop(0, CH, _z, 0)
        for z in range(rows_per_tile // CH):
            pltpu.sync_copy(bufa, acc.at[pl.ds(s * rows_per_tile + z * CH, CH)])
        plsc.subcore_barrier()

        def _sc_body(sc, _):
            pltpu.sync_copy(col_hbm.at[k, t, pl.ds(sc * SCH, SCH)], colb)
            pltpu.sync_copy(w_hbm.at[k, t, pl.ds(sc * SCH * CH, SCH * CH)], wb)

            def _chunk(q, _):
                def _g(g, _):
                    wvec = wb[pl.ds(q * CH + g * 16, 16)]
                    for i in range(16):
                        bufa[g * 16 + i, pl.ds(0, 16)] = wvec[i] * ones16
                    return 0
                lax.fori_loop(0, CH // 16, _g, 0)
                pltpu.sync_copy(bufa, acc.at[colb.at[q]], add=True)
                return 0

            lax.fori_loop(0, SCH, _chunk, 0)
            return 0

        lax.fori_loop(0, NCHUNK // SCH, _sc_body, 0)
        plsc.subcore_barrier()
        for z in range(rows_per_tile // CH):
            off = s * rows_per_tile + z * CH
            pltpu.sync_copy(acc.at[pl.ds(off, CH)],
                            out_hbm.at[c, k, pl.ds(off, CH)])


def _conv_body(gt_hbm, row2_hbm, col_hbm, w_hbm, out_hbm,
               acc, rowb, colb, wb, bufa, bufb, semA, semB, semSA, semSB):
    c = lax.axis_index("c")
    s = lax.axis_index("s")
    t = c * 16 + s
    rows_per_tile = NP // 16
    zeros16 = jnp.zeros((16,), jnp.float32)

    def _zero_bufa():
        def _z(i, _):
            for sub in range(8):
                bufa[i, pl.ds(sub * 16, 16)] = zeros16
            return 0
        lax.fori_loop(0, CH, _z, 0)

    def _scale(buf, q):
        def _g(g, _):
            wvec = wb[pl.ds(q * CH + g * 16, 16)]
            for i in range(16):
                wv = wvec[i]
                e = g * 16 + i
                for sub in range(8):
                    sl = pl.ds(sub * 16, 16)
                    buf[e, sl] = buf[e, sl] * wv
            return 0

        lax.fori_loop(0, CH // 16, _g, 0)

    for k in range(3):
        _zero_bufa()
        for z in range(rows_per_tile // CH):
            pltpu.sync_copy(bufa, acc.at[pl.ds(s * rows_per_tile + z * CH, CH)])
        plsc.subcore_barrier()

        def _sc_body(sc, _):
            pltpu.sync_copy(row2_hbm.at[k, t, pl.ds(sc * SCH, SCH)], rowb)
            pltpu.sync_copy(col_hbm.at[k, t, pl.ds(sc * SCH, SCH)], colb)
            pltpu.sync_copy(w_hbm.at[k, t, pl.ds(sc * SCH * CH, SCH * CH)], wb)

            def _gather(q, buf, sem):
                pltpu.async_copy(gt_hbm.at[rowb.at[q, pl.ds(0, 64)]],
                                 buf.at[pl.ds(0, 64)], sem)
                pltpu.async_copy(gt_hbm.at[rowb.at[q, pl.ds(64, 64)]],
                                 buf.at[pl.ds(64, 64)], sem)

            def _gwait(buf, sem):
                pltpu.make_async_copy(gt_hbm.at[rowb.at[0]], buf, sem).wait()

            _gather(0, bufa, semA)
            _gather(1, bufb, semB)

            def _pair(j, _):
                qa = 2 * j
                qb = 2 * j + 1
                _gwait(bufa, semA)
                if k > 0:
                    _scale(bufa, qa)
                sa = pltpu.async_copy(bufa, acc.at[colb.at[qa]], semSA,
                                      add=True)
                _gwait(bufb, semB)
                sa.wait()
                _gather(qa + 2, bufa, semA)
                if k > 0:
                    _scale(bufb, qb)
                sb = pltpu.async_copy(bufb, acc.at[colb.at[qb]], semSB,
                                      add=True)
                sb.wait()
                _gather(qb + 2, bufb, semB)
                return 0

            lax.fori_loop(0, SCH // 2 - 1, _pair, 0)

            qa = SCH - 2
            qb = SCH - 1
            _gwait(bufa, semA)
            if k > 0:
                _scale(bufa, qa)
            sa = pltpu.async_copy(bufa, acc.at[colb.at[qa]], semSA, add=True)
            _gwait(bufb, semB)
            sa.wait()
            if k > 0:
                _scale(bufb, qb)
            sb = pltpu.async_copy(bufb, acc.at[colb.at[qb]], semSB, add=True)
            sb.wait()
            return 0

        lax.fori_loop(0, NCHUNK // SCH, _sc_body, 0)
        plsc.subcore_barrier()
        for z in range(rows_per_tile // CH):
            off = s * rows_per_tile + z * CH
            pltpu.sync_copy(acc.at[pl.ds(off, CH)],
                            out_hbm.at[c, k, pl.ds(off, CH)])


def _mm1_body(x_ref, w1t_ref, sdeg_ref, g0_ref, g1_ref, g2_ref, dinv_ref):
    h = jnp.dot(x_ref[...], w1t_ref[...])
    gs = (g0_ref, g1_ref, g2_ref)
    dcols = []
    for k in range(3):
        deg = sdeg_ref[0, k, :, 0:1] + sdeg_ref[1, k, :, 0:1] + 1.0
        dinv = jnp.where(deg > 0, lax.rsqrt(deg), 0.0)
        dcols.append(dinv)
        gs[k][...] = h * dinv
    dinv_ref[...] = jnp.concatenate(
        dcols + [jnp.zeros((dcols[0].shape[0], 5), dcols[0].dtype)], axis=1)


def _mm2_body(S_ref, g0_ref, g1_ref, g2_ref, dinv_ref, b_ref, w2t_ref,
              o0_ref, o1_ref, o2_ref):
    dinv = dinv_ref[...]
    b = b_ref[...]
    gs = (g0_ref[...], g1_ref[...], g2_ref[...])
    h2 = None
    for k in range(3):
        a = jnp.maximum(
            dinv[:, k:k + 1] * (S_ref[0, k] + S_ref[1, k] + gs[k]) + b, 0.0)
        p = jnp.dot(a, w2t_ref[pl.ds(k * D, D), :])
        h2 = p if h2 is None else h2 + p
    o0_ref[...] = h2 * dinv[:, 0:1]
    o1_ref[...] = h2 * dinv[:, 1:2]
    o2_ref[...] = h2 * dinv[:, 2:3]


def _mm3_body(Q_ref, g0_ref, g1_ref, g2_ref, dinv_ref, b_ref, wct_ref,
              bc_ref, out_ref):
    dinv = dinv_ref[...]
    b = b_ref[...]
    gs = (g0_ref[...], g1_ref[...], g2_ref[...])
    logits = None
    for k in range(3):
        a = jnp.maximum(
            dinv[:, k:k + 1] * (Q_ref[0, k] + Q_ref[1, k] + gs[k]) + b, 0.0)
        p = jnp.dot(a, wct_ref[pl.ds(k * D, D), :])
        logits = p if logits is None else logits + p
    logits = logits + bc_ref[...]
    m = jnp.max(logits, axis=1, keepdims=True)
    sh = logits - m
    lse = jnp.log(jnp.sum(jnp.exp(sh), axis=1, keepdims=True))
    out_ref[...] = sh - lse


_SC_KERNELS = None


def _get_sc_kernels():
    global _SC_KERNELS
    if _SC_KERNELS is None:
        mesh = plsc.VectorSubcoreMesh(core_axis_name="c", subcore_axis_name="s")
        deg = functools.partial(
            pl.kernel,
            mesh=mesh,
            out_type=jax.ShapeDtypeStruct((2, 3, NP, D), jnp.float32),
            scratch_types=[
                pltpu.VMEM_SHARED((NP, D), jnp.float32),
                pltpu.VMEM((SCH, CH), jnp.int32),
                pltpu.VMEM((SCH * CH,), jnp.float32),
                pltpu.VMEM((CH, D), jnp.float32),
            ],
        )(_deg_body)
        conv = functools.partial(
            pl.kernel,
            mesh=mesh,
            out_type=jax.ShapeDtypeStruct((2, 3, NP, D), jnp.float32),
            scratch_types=[
                pltpu.VMEM_SHARED((NP, D), jnp.float32),
                pltpu.VMEM((SCH, CH), jnp.int32),
                pltpu.VMEM((SCH, CH), jnp.int32),
                pltpu.VMEM((SCH * CH,), jnp.float32),
                pltpu.VMEM((CH, D), jnp.float32),
                pltpu.VMEM((CH, D), jnp.float32),
                pltpu.SemaphoreType.DMA,
                pltpu.SemaphoreType.DMA,
                pltpu.SemaphoreType.DMA,
                pltpu.SemaphoreType.DMA,
            ],
        )(_conv_body)
        _SC_KERNELS = (deg, conv)
    return _SC_KERNELS


BLK = 1024
GRID = NP // BLK


def _row_spec(width):
    return pl.BlockSpec((BLK, width), lambda i: (i, 0))


def _full_spec(shape):
    return pl.BlockSpec(shape, lambda i: tuple(0 for _ in shape))


def kernel(x, edge_index, edge_in, edge_out, in_w, out_w, W1, b1, W2, b2, Wc, bc):
    f32 = jnp.float32
    i32 = jnp.int32

    ones_w = jnp.ones((E,), f32)
    rows = jnp.stack([edge_index[0], edge_in[0], edge_out[0]]).astype(i32)
    cols = jnp.stack([edge_index[1], edge_in[1], edge_out[1]]).astype(i32)
    ws = jnp.stack([ones_w, in_w.astype(f32), out_w.astype(f32)])
    rows = rows + (jnp.arange(3, dtype=i32) * NP)[:, None]

    pad = NT * NCHUNK * CH - E
    rows_p = jnp.pad(rows.reshape(3, NT, EPT), ((0, 0), (0, 0), (0, pad // NT)),
                     constant_values=0).reshape(3, NT, NCHUNK, CH)
    cols_p = jnp.pad(cols.reshape(3, NT, EPT), ((0, 0), (0, 0), (0, pad // NT)),
                     constant_values=PAD_COL).reshape(3, NT, NCHUNK, CH)
    ws_p = jnp.pad(ws.reshape(3, NT, EPT), ((0, 0), (0, 0), (0, pad // NT)),
                   constant_values=0.0).reshape(3, NT, NCHUNK, CH)

    x_p = jnp.pad(x.astype(f32), ((0, NP - N), (0, 0)))
    W1T = W1.astype(f32).T
    W2T = W2.astype(f32).T
    WcT = Wc.astype(f32).T
    OUT = Wc.shape[0]
    bc2 = bc.astype(f32)[None, :]

    ws_flat = ws_p.reshape(3, NT, NCHUNK * CH)

    _deg_kernel, _conv_kernel = _get_sc_kernels()
    Sdeg = _deg_kernel(cols_p, ws_flat)


    g0, g1, g2, dinv = pl.pallas_call(
        _mm1_body,
        grid=(GRID,),
        in_specs=[
            _row_spec(D),
            _full_spec((D, D)),
            pl.BlockSpec((2, 3, BLK, D), lambda i: (0, 0, i, 0)),
        ],
        out_specs=[_row_spec(D), _row_spec(D), _row_spec(D), _row_spec(8)],
        out_shape=[
            jax.ShapeDtypeStruct((NP, D), f32),
            jax.ShapeDtypeStruct((NP, D), f32),
            jax.ShapeDtypeStruct((NP, D), f32),
            jax.ShapeDtypeStruct((NP, 8), f32),
        ],
    )(x_p, W1T, Sdeg)

    gt = jnp.concatenate([g0, g1, g2], axis=0)

    S = _conv_kernel(gt, rows_p, cols_p, ws_flat)

    h0, h1, h2 = pl.pallas_call(
        _mm2_body,
        grid=(GRID,),
        in_specs=[
            pl.BlockSpec((2, 3, BLK, D), lambda i: (0, 0, i, 0)),
            _row_spec(D), _row_spec(D), _row_spec(D), _row_spec(8),
            _full_spec((1, D)),
            _full_spec((3 * D, D)),
        ],
        out_specs=[_row_spec(D), _row_spec(D), _row_spec(D)],
        out_shape=[
            jax.ShapeDtypeStruct((NP, D), f32),
            jax.ShapeDtypeStruct((NP, D), f32),
            jax.ShapeDtypeStruct((NP, D), f32),
        ],
    )(S, g0, g1, g2, dinv, b1.astype(f32), W2T)

    gt2 = jnp.concatenate([h0, h1, h2], axis=0)

    Q = _conv_kernel(gt2, rows_p, cols_p, ws_flat)

    out = pl.pallas_call(
        _mm3_body,
        grid=(GRID,),
        in_specs=[
            pl.BlockSpec((2, 3, BLK, D), lambda i: (0, 0, i, 0)),
            _row_spec(D), _row_spec(D), _row_spec(D), _row_spec(8),
            _full_spec((1, D)),
            _full_spec((3 * D, OUT)),
            _full_spec((1, OUT)),
        ],
        out_specs=_row_spec(OUT),
        out_shape=jax.ShapeDtypeStruct((NP, OUT), f32),
    )(Q, h0, h1, h2, dinv, b2.astype(f32), WcT, bc2)

    return out[:N]
